# Initial kernel scaffold; baseline (speedup 1.0000x reference)
import jax
import jax.numpy as jnp
from jax.experimental import pallas as pl


def kernel(x, embedding_weight, rnn_weight, rnn_bias):
    raise NotImplementedError("write your pallas kernel here")


# speedup vs baseline: 4.4172x; 4.4172x over previous
import functools

import jax
import jax.numpy as jnp
from jax import lax
from jax.experimental import pallas as pl
from jax.experimental.pallas import tpu as pltpu
from jax.experimental.pallas import tpu_sc as plsc

_B, _L = 16384, 200
_V, _D = 1000, 10
_N = _B * _L
_NC, _NS = 2, 16
_NW = _NC * _NS
_PER_W = _N // _NW
_CHUNK = 2048
_NCHUNK = _PER_W // _CHUNK
_STEPS = _CHUNK // 16


def _fold_table_tc(emb, w, b):
    def body(e_ref, w_ref, b_ref, o_ref):
        o_ref[...] = (
            jnp.dot(e_ref[...], w_ref[...].T, preferred_element_type=jnp.float32)
            + b_ref[...]
        )

    return pl.pallas_call(
        body,
        out_shape=jax.ShapeDtypeStruct((_V, _D), jnp.float32),
    )(emb, w, b.reshape(1, _D))


def _gather_sc(table_flat, idx_flat):
    mesh = plsc.VectorSubcoreMesh(core_axis_name="c", subcore_axis_name="s")

    @functools.partial(
        pl.kernel,
        mesh=mesh,
        out_type=jax.ShapeDtypeStruct((_N * _D,), jnp.float32),
        scratch_types=[
            pltpu.VMEM((_V * _D,), jnp.float32),
            pltpu.VMEM((_CHUNK,), jnp.int32),
            pltpu.VMEM((_CHUNK * _D,), jnp.float32),
        ],
        compiler_params=pltpu.CompilerParams(needs_layout_passes=False),
    )
    def k(table_hbm, idx_hbm, out_hbm, table_v, idx_v, out_v):
        wid = lax.axis_index("s") * _NC + lax.axis_index("c")
        pltpu.sync_copy(table_hbm, table_v)
        i10 = lax.iota(jnp.int32, 16) * _D
        base = wid * _PER_W

        def chunk_body(c, carry):
            off = base + c * _CHUNK
            pltpu.sync_copy(idx_hbm.at[pl.ds(off, _CHUNK)], idx_v)

            def jbody(j, carry2):
                iv = idx_v[pl.ds(j * 16, 16)]
                rb = iv * _D
                ob = i10 + j * (16 * _D)
                for dd in range(_D):
                    vals = plsc.load_gather(table_v, [rb + dd])
                    plsc.store_scatter(out_v, [ob + dd], vals)
                return carry2

            lax.fori_loop(0, _STEPS, jbody, 0)
            pltpu.sync_copy(out_v, out_hbm.at[pl.ds(off * _D, _CHUNK * _D)])
            return carry

        lax.fori_loop(0, _NCHUNK, chunk_body, 0)

    return k(table_flat, idx_flat)


def kernel(x, embedding_weight, rnn_weight, rnn_bias):
    t = _fold_table_tc(embedding_weight, rnn_weight, rnn_bias)
    idx = x.reshape(-1).astype(jnp.int32)
    out = _gather_sc(t.reshape(-1), idx)
    return out.reshape(_B, _L, _D)


# speedup vs baseline: 6.0734x; 1.3749x over previous
import functools

import jax
import jax.numpy as jnp
from jax import lax
from jax.experimental import pallas as pl
from jax.experimental.pallas import tpu as pltpu
from jax.experimental.pallas import tpu_sc as plsc

_B, _L = 16384, 200
_V, _D = 1000, 10
_N = _B * _L
_NC, _NS = 2, 16
_NW = _NC * _NS
_PER_W = _N // _NW
_CHUNK = 512
_NCHUNK = _PER_W // _CHUNK
_STEPS = _CHUNK // 16


def _fold_table_tc(emb, w, b):
    def body(e_ref, w_ref, b_ref, o_ref):
        o_ref[...] = (
            jnp.dot(e_ref[...], w_ref[...].T, preferred_element_type=jnp.float32)
            + b_ref[...]
        )

    return pl.pallas_call(
        body,
        out_shape=jax.ShapeDtypeStruct((_V, _D), jnp.float32),
    )(emb, w, b.reshape(1, _D))


def _gather_sc(table_flat, idx_flat):
    mesh = plsc.VectorSubcoreMesh(core_axis_name="c", subcore_axis_name="s")

    @functools.partial(
        pl.kernel,
        mesh=mesh,
        out_type=jax.ShapeDtypeStruct((_N, _D), jnp.float32),
        scratch_types=[
            pltpu.VMEM((_V * _D,), jnp.float32),
            pltpu.VMEM((_CHUNK,), jnp.int32),
            pltpu.VMEM((_CHUNK, _D), jnp.float32),
        ],
        compiler_params=pltpu.CompilerParams(needs_layout_passes=False),
    )
    def k(table_hbm, idx_hbm, out_hbm, table_v, idx_v, out_v):
        wid = lax.axis_index("s") * _NC + lax.axis_index("c")
        pltpu.sync_copy(table_hbm, table_v)
        ii = lax.iota(jnp.int32, 16)
        base = wid * _PER_W

        def chunk_body(c, carry):
            off = base + c * _CHUNK
            pltpu.sync_copy(idx_hbm.at[pl.ds(off, _CHUNK)], idx_v)

            def jbody(j, carry2):
                iv = idx_v[pl.ds(j * 16, 16)]
                rb = iv * _D
                orow = ii + j * 16
                for dd in range(_D):
                    vals = plsc.load_gather(table_v, [rb + dd])
                    plsc.store_scatter(
                        out_v, [orow, jnp.full((16,), dd, jnp.int32)], vals
                    )
                return carry2

            lax.fori_loop(0, _STEPS, jbody, 0)
            pltpu.sync_copy(out_v, out_hbm.at[pl.ds(off, _CHUNK), :])
            return carry

        lax.fori_loop(0, _NCHUNK, chunk_body, 0)

    return k(table_flat, idx_flat)


def kernel(x, embedding_weight, rnn_weight, rnn_bias):
    t = _fold_table_tc(embedding_weight, rnn_weight, rnn_bias)
    idx = x.reshape(-1).astype(jnp.int32)
    out = _gather_sc(t.reshape(-1), idx)
    return out.reshape(_B, _L, _D)
